# baseline (speedup 1.0000x reference)
import functools

import jax
import jax.numpy as jnp
from jax import lax
from jax.experimental import pallas as pl
from jax.experimental.pallas import tpu as pltpu
from jax.experimental.pallas import tpu_sc as plsc

VOCAB = 1002
N_D = 64
BATCH = 4096
HIST = 200

NC = 2
NS = 16
NW = NC * NS

L = 16
NCH = N_D // L
BPT = BATCH // NW
NBG = BPT // L
SROW = N_D + 1


def _emb_body(idxt_hbm, table_hbm, out_hbm, table_v, idx_v, stage_v, blk_v,
              sem0, sem1):
    wid = lax.axis_index("s") * NC + lax.axis_index("c")
    b0 = wid * BPT
    sems = (sem0, sem1)

    pltpu.sync_copy(table_hbm, table_v)
    pltpu.sync_copy(idxt_hbm.at[:, pl.ds(b0, BPT)], idx_v)

    def dyn_gather(vec, idx16):
        return lax.gather(
            vec, idx16[:, None],
            lax.GatherDimensionNumbers(
                offset_dims=(), collapsed_slice_dims=(0,),
                start_index_map=(0,)),
            (1,), mode=lax.GatherScatterMode.PROMISE_IN_BOUNDS)

    iota = lax.iota(jnp.int32, L)
    iota_c = [iota + c * L for c in range(NCH)]
    iota_s = iota * SROW

    def fill_block(h, b):
        @plsc.parallel_loop(0, NBG, 1, unroll=2)
        def _group(g):
            idxv = idx_v[h, pl.ds(g * L, L)]
            addr = idxv * N_D
            gbase = g * (L * SROW)
            for r in range(L):
                src0 = dyn_gather(addr, jnp.full((L,), r, jnp.int32))
                for c in range(NCH):
                    v = plsc.load_gather(table_v, [src0 + iota_c[c]])
                    stage_v[pl.ds(gbase + r * SROW + c * L, L)] = v
            base = iota_s + gbase
            for d in range(N_D):
                v = plsc.load_gather(stage_v, [base + d])
                blk_v[b, d, pl.ds(g * L, L)] = v

    def body(i, carry):
        descs = {}
        for kb in range(2):
            h = i * 2 + kb
            fill_block(h, kb)
            descs[kb] = pltpu.async_copy(
                blk_v.at[kb], out_hbm.at[h, :, pl.ds(b0, BPT)], sems[kb]
            )
        descs[0].wait()
        descs[1].wait()
        return carry

    lax.fori_loop(0, HIST // 2, body, 0)


@jax.jit
def _embedding_sc(idx_t, table_flat):
    mesh = plsc.VectorSubcoreMesh(
        core_axis_name="c", subcore_axis_name="s",
        num_cores=NC, num_subcores=NS,
    )
    f = functools.partial(
        pl.kernel,
        out_type=jax.ShapeDtypeStruct((HIST, N_D, BATCH), jnp.float32),
        mesh=mesh,
        scratch_types=[
            pltpu.VMEM((VOCAB * N_D,), jnp.float32),
            pltpu.VMEM((HIST, BPT), jnp.int32),
            pltpu.VMEM((BPT * SROW,), jnp.float32),
            pltpu.VMEM((2, N_D, BPT), jnp.float32),
            pltpu.SemaphoreType.DMA,
            pltpu.SemaphoreType.DMA,
        ],
        compiler_params=pltpu.CompilerParams(
            use_tc_tiling_on_sc=True, needs_layout_passes=False),
    )(_emb_body)
    return f(idx_t, table_flat)


def kernel(input, table):
    idx_t = jnp.transpose(input.astype(jnp.int32))
    out_t = _embedding_sc(idx_t, table.reshape(-1))
    return jnp.transpose(out_t, (2, 0, 1))


# speedup vs baseline: 2.2892x; 2.2892x over previous
import functools

import jax
import jax.numpy as jnp
from jax import lax
from jax.experimental import pallas as pl
from jax.experimental.pallas import tpu as pltpu
from jax.experimental.pallas import tpu_sc as plsc

VOCAB = 1002
N_D = 64
BATCH = 4096
HIST = 200

NC = 2
NS = 16
NW = NC * NS

L = 16
BPT = BATCH // NW
NBG = BPT // L
TSTRIDE = 1008


def _emb_body(idxt_hbm, tablet_hbm, out_hbm, table_v, idx_v, blk_v,
              sem0, sem1):
    wid = lax.axis_index("s") * NC + lax.axis_index("c")
    b0 = wid * BPT
    sems = (sem0, sem1)

    pltpu.sync_copy(tablet_hbm, table_v)
    pltpu.sync_copy(idxt_hbm.at[:, pl.ds(b0, BPT)], idx_v)

    def fill_block(h, b):
        @plsc.parallel_loop(0, NBG, 1, unroll=2)
        def _group(g):
            idxv = idx_v[h, pl.ds(g * L, L)]
            for d in range(N_D):
                v = plsc.load_gather(table_v, [idxv + d * TSTRIDE])
                blk_v[b, d, pl.ds(g * L, L)] = v

    def body(i, carry):
        descs = {}
        for kb in range(2):
            h = i * 2 + kb
            fill_block(h, kb)
            descs[kb] = pltpu.async_copy(
                blk_v.at[kb], out_hbm.at[h, :, pl.ds(b0, BPT)], sems[kb]
            )
        descs[0].wait()
        descs[1].wait()
        return carry

    lax.fori_loop(0, HIST // 2, body, 0)


@jax.jit
def _embedding_sc(idx_t, table_t):
    mesh = plsc.VectorSubcoreMesh(
        core_axis_name="c", subcore_axis_name="s",
        num_cores=NC, num_subcores=NS,
    )
    f = functools.partial(
        pl.kernel,
        out_type=jax.ShapeDtypeStruct((HIST, N_D, BATCH), jnp.float32),
        mesh=mesh,
        scratch_types=[
            pltpu.VMEM((N_D * TSTRIDE,), jnp.float32),
            pltpu.VMEM((HIST, BPT), jnp.int32),
            pltpu.VMEM((2, N_D, BPT), jnp.float32),
            pltpu.SemaphoreType.DMA,
            pltpu.SemaphoreType.DMA,
        ],
        compiler_params=pltpu.CompilerParams(
            use_tc_tiling_on_sc=True, needs_layout_passes=False),
    )(_emb_body)
    return f(idx_t, table_t)


def kernel(input, table):
    idx_t = jnp.transpose(input.astype(jnp.int32))
    table_t = jnp.pad(jnp.transpose(table),
                      ((0, 0), (0, TSTRIDE - VOCAB))).reshape(-1)
    out_t = _embedding_sc(idx_t, table_t)
    return jnp.transpose(out_t, (2, 0, 1))


# speedup vs baseline: 3.9822x; 1.7395x over previous
import functools

import jax
import jax.numpy as jnp
from jax import lax
from jax.experimental import pallas as pl
from jax.experimental.pallas import tpu as pltpu
from jax.experimental.pallas import tpu_sc as plsc

VOCAB = 1002
N_D = 64
BATCH = 4096
HIST = 200

NC = 2
NS = 16
NW = NC * NS

L = 16
BPT = BATCH // NW
NBG = BPT // L
TSTRIDE = 1008


def _emb_body(idxt_hbm, tablet_hbm, out_hbm, table_v, idx_v, blk_v,
              sem0, sem1):
    wid = lax.axis_index("s") * NC + lax.axis_index("c")
    b0 = wid * BPT
    sems = (sem0, sem1)

    pltpu.sync_copy(tablet_hbm, table_v)
    pltpu.sync_copy(idxt_hbm.at[:, pl.ds(b0, BPT)], idx_v)

    def fill_block(h, b):
        @plsc.parallel_loop(0, NBG, 1, unroll=4)
        def _group(g):
            idxv = idx_v[h, pl.ds(g * L, L)]
            for d in range(N_D):
                v = plsc.load_gather(table_v, [idxv + d * TSTRIDE])
                blk_v[b, d, pl.ds(g * L, L)] = v

    def body(i, carry):
        descs = {}
        for kb in range(2):
            h = i * 2 + kb
            fill_block(h, kb)
            descs[kb] = pltpu.async_copy(
                blk_v.at[kb], out_hbm.at[h, :, pl.ds(b0, BPT)], sems[kb]
            )
        descs[0].wait()
        descs[1].wait()
        return carry

    lax.fori_loop(0, HIST // 2, body, 0)


@jax.jit
def _embedding_sc(idx_t, table_t):
    mesh = plsc.VectorSubcoreMesh(
        core_axis_name="c", subcore_axis_name="s",
        num_cores=NC, num_subcores=NS,
    )
    f = functools.partial(
        pl.kernel,
        out_type=jax.ShapeDtypeStruct((HIST, N_D, BATCH), jnp.float32),
        mesh=mesh,
        scratch_types=[
            pltpu.VMEM((N_D * TSTRIDE,), jnp.float32),
            pltpu.VMEM((HIST, BPT), jnp.int32),
            pltpu.VMEM((2, N_D, BPT), jnp.float32),
            pltpu.SemaphoreType.DMA,
            pltpu.SemaphoreType.DMA,
        ],
        compiler_params=pltpu.CompilerParams(
            use_tc_tiling_on_sc=True, needs_layout_passes=False),
    )(_emb_body)
    return f(idx_t, table_t)


def kernel(input, table):
    idx_t = jnp.transpose(input.astype(jnp.int32))
    table_t = jnp.pad(jnp.transpose(table),
                      ((0, 0), (0, TSTRIDE - VOCAB))).reshape(-1)
    out_t = _embedding_sc(idx_t, table_t)
    return jnp.transpose(out_t, (2, 0, 1))
